# baseline (speedup 1.0000x reference)
import functools

import jax
import jax.numpy as jnp
from jax import lax
from jax.experimental import pallas as pl
from jax.experimental.pallas import tpu as pltpu
from jax.experimental.pallas import tpu_sc as plsc

_MULT = 32
_NIR = 9
_NWC = 3
_WIDX = (0, 1, 1, 1, 2, 2, 2, 2, 2)
_OUTC = _MULT * _NIR
_WC = _MULT * _NWC


@functools.cache
def _build(E: int, C: int):
    NC, NS, L = 2, 16, 16
    NW = NC * NS
    EPW = E // NW
    n_chunks = EPW // C
    mesh = plsc.VectorSubcoreMesh(
        core_axis_name="c", subcore_axis_name="s",
        num_cores=NC, num_subcores=NS)

    @functools.partial(
        pl.kernel,
        out_type=jax.ShapeDtypeStruct((E * _OUTC,), jnp.float32),
        mesh=mesh,
        compiler_params=pltpu.CompilerParams(needs_layout_passes=False),
        scratch_types=[
            pltpu.VMEM((C * _NIR,), jnp.float32),
            pltpu.VMEM((C * _WC,), jnp.float32),
            pltpu.VMEM((C * _OUTC,), jnp.float32),
        ],
    )
    def k(a_hbm, w_hbm, out_hbm, a_v, w_v, o_v):
        wid = lax.axis_index("s") * NC + lax.axis_index("c")
        base = wid * EPW
        lanes = lax.iota(jnp.int32, L)

        def chunk_body(ci, carry):
            e0 = base + ci * C
            pltpu.sync_copy(a_hbm.at[pl.ds(e0 * _NIR, C * _NIR)], a_v)
            pltpu.sync_copy(w_hbm.at[pl.ds(e0 * _WC, C * _WC)], w_v)

            def group_body(g, carry2):
                r = g * L + lanes
                ra = r * _NIR
                rw = r * _WC
                ro = r * _OUTC
                a_cols = [
                    plsc.load_gather(a_v, [ra + i]) for i in range(_NIR)
                ]
                for u in range(_MULT):
                    w_cols = [
                        plsc.load_gather(w_v, [rw + (u * _NWC + kk)])
                        for kk in range(_NWC)
                    ]
                    for i in range(_NIR):
                        plsc.store_scatter(
                            o_v, [ro + (u * _NIR + i)],
                            a_cols[i] * w_cols[_WIDX[i]])
                return carry2

            lax.fori_loop(0, C // L, group_body, 0, unroll=False)
            pltpu.sync_copy(o_v, out_hbm.at[pl.ds(e0 * _OUTC, C * _OUTC)])
            return carry

        lax.fori_loop(0, n_chunks, chunk_body, 0, unroll=False)

    return k


def _pick_chunk(EPW: int) -> int:
    best = 0
    for c in range(16, 161, 16):
        if EPW % c == 0:
            best = c
    return best


def kernel(edge_attr, weights):
    E = edge_attr.shape[0]
    EPW = E // 32
    C = _pick_chunk(EPW)
    out = _build(E, C)(edge_attr.reshape(-1), weights.reshape(-1))
    return out.reshape(E, _MULT, _NIR)


# speedup vs baseline: 27.0883x; 27.0883x over previous
import functools

import jax
import jax.numpy as jnp
from jax import lax
from jax.experimental import pallas as pl
from jax.experimental.pallas import tpu as pltpu
from jax.experimental.pallas import tpu_sc as plsc

_MULT = 32
_NIR = 9
_NWC = 3
_WIDX = (0, 1, 1, 1, 2, 2, 2, 2, 2)
_WC = _MULT * _NWC
_Z = 128


@functools.cache
def _build(E: int):
    NC, NS, L = 2, 16, 16
    NW = NC * NS
    TCOLS = E // _Z
    mesh = plsc.VectorSubcoreMesh(
        core_axis_name="c", subcore_axis_name="s",
        num_cores=NC, num_subcores=NS)

    @functools.partial(
        pl.kernel,
        out_type=jax.ShapeDtypeStruct((_NIR, _MULT, E), jnp.float32),
        mesh=mesh,
        compiler_params=pltpu.CompilerParams(needs_layout_passes=False),
        scratch_types=[
            pltpu.VMEM((2, _NIR, _Z), jnp.float32),
            pltpu.VMEM((2, _WC, _Z), jnp.float32),
            pltpu.VMEM((2, _NIR, _MULT, _Z), jnp.float32),
            pltpu.SemaphoreType.DMA,
            pltpu.SemaphoreType.DMA,
            pltpu.SemaphoreType.DMA,
            pltpu.SemaphoreType.DMA,
            pltpu.SemaphoreType.DMA,
            pltpu.SemaphoreType.DMA,
        ],
    )
    def k(a_hbm, w_hbm, out_hbm, a_v, w_v, o_v,
          sa0, sa1, sw0, sw1, so0, so1):
        sa = (sa0, sa1)
        sw = (sw0, sw1)
        so = (so0, so1)
        wid = lax.axis_index("s") * NC + lax.axis_index("c")
        lo = wid * TCOLS // NW
        hi = (wid + 1) * TCOLS // NW

        def start_in(col, b):
            z0 = col * _Z
            pltpu.async_copy(a_hbm.at[:, pl.ds(z0, _Z)], a_v.at[b], sa[b])
            pltpu.async_copy(w_hbm.at[:, pl.ds(z0, _Z)], w_v.at[b], sw[b])

        def wait_in(b):
            pltpu.make_async_copy(a_hbm.at[:, pl.ds(0, _Z)],
                                  a_v.at[b], sa[b]).wait()
            pltpu.make_async_copy(w_hbm.at[:, pl.ds(0, _Z)],
                                  w_v.at[b], sw[b]).wait()

        def start_out(col, b):
            z0 = col * _Z
            pltpu.async_copy(o_v.at[b],
                             out_hbm.at[:, :, pl.ds(z0, _Z)], so[b])

        def wait_out(b):
            pltpu.make_async_copy(o_v.at[b],
                                  out_hbm.at[:, :, pl.ds(0, _Z)],
                                  so[b]).wait()

        def compute(b):
            a_vb = a_v.at[b]
            w_vb = w_v.at[b]
            o_vb = o_v.at[b]

            def pos_body(pos, carry):
                p = pos * L
                a_regs = [a_vb.at[i][pl.ds(p, L)] for i in range(_NIR)]
                for u in range(_MULT):
                    w_regs = [w_vb.at[u * _NWC + kk][pl.ds(p, L)]
                              for kk in range(_NWC)]
                    for i in range(_NIR):
                        o_vb.at[i].at[u][pl.ds(p, L)] = (
                            a_regs[i] * w_regs[_WIDX[i]])
                return carry

            lax.fori_loop(0, _Z // L, pos_body, 0, unroll=False)

        ncols = hi - lo
        start_in(lo, 0)

        def body(it, carry):
            for b in range(2):
                ci = it * 2 + b
                col = lo + ci

                @pl.when(ci < ncols)
                def _():
                    wait_in(b)

                    @pl.when(ci + 1 < ncols)
                    def _():
                        start_in(col + 1, 1 - b)

                    @pl.when(ci >= 2)
                    def _():
                        wait_out(b)

                    compute(b)
                    start_out(col, b)
            return carry

        lax.fori_loop(0, (ncols + 1) // 2, body, 0, unroll=False)

        even = (ncols % 2) == 0

        @pl.when(even)
        def _():
            wait_out(0)
            wait_out(1)

        @pl.when(jnp.logical_not(even))
        def _():
            wait_out(1)
            wait_out(0)

    return k


def kernel(edge_attr, weights):
    E = edge_attr.shape[0]
    a_t = edge_attr.T
    w_t = weights.T
    out_t = _build(E)(a_t, w_t)
    return out_t.transpose(2, 1, 0)


# speedup vs baseline: 46.2043x; 1.7057x over previous
import functools

import jax
import jax.numpy as jnp
from jax import lax
from jax.experimental import pallas as pl
from jax.experimental.pallas import tpu as pltpu
from jax.experimental.pallas import tpu_sc as plsc

_MULT = 32
_NIR = 9
_NWC = 3
_WIDX = (0, 1, 1, 1, 2, 2, 2, 2, 2)
_WC = _MULT * _NWC
_Z = 128


@functools.cache
def _build(E: int):
    NC, NS, L = 2, 16, 16
    NW = NC * NS
    TCOLS = E // _Z
    mesh = plsc.VectorSubcoreMesh(
        core_axis_name="c", subcore_axis_name="s",
        num_cores=NC, num_subcores=NS)

    @functools.partial(
        pl.kernel,
        out_type=jax.ShapeDtypeStruct((_NIR, _MULT, E), jnp.float32),
        mesh=mesh,
        compiler_params=pltpu.CompilerParams(needs_layout_passes=False),
        scratch_types=[
            pltpu.VMEM((2, _NIR, _Z), jnp.float32),
            pltpu.VMEM((2, _WC, _Z), jnp.float32),
            pltpu.VMEM((2, _NIR, _MULT, _Z), jnp.float32),
            pltpu.SemaphoreType.DMA,
            pltpu.SemaphoreType.DMA,
            pltpu.SemaphoreType.DMA,
            pltpu.SemaphoreType.DMA,
            pltpu.SemaphoreType.DMA,
            pltpu.SemaphoreType.DMA,
        ],
    )
    def k(a_hbm, w_hbm, out_hbm, a_v, w_v, o_v,
          sa0, sa1, sw0, sw1, so0, so1):
        sa = (sa0, sa1)
        sw = (sw0, sw1)
        so = (so0, so1)
        wid = lax.axis_index("s") * NC + lax.axis_index("c")
        lo = wid * TCOLS // NW
        hi = (wid + 1) * TCOLS // NW

        def start_in(col, b):
            z0 = col * _Z
            pltpu.async_copy(a_hbm.at[:, pl.ds(z0, _Z)], a_v.at[b], sa[b])
            pltpu.async_copy(w_hbm.at[:, pl.ds(z0, _Z)], w_v.at[b], sw[b])

        def wait_in(b):
            pltpu.make_async_copy(a_hbm.at[:, pl.ds(0, _Z)],
                                  a_v.at[b], sa[b]).wait()
            pltpu.make_async_copy(w_hbm.at[:, pl.ds(0, _Z)],
                                  w_v.at[b], sw[b]).wait()

        def start_out(col, b):
            z0 = col * _Z
            pltpu.async_copy(o_v.at[b],
                             out_hbm.at[:, :, pl.ds(z0, _Z)], so[b])

        def wait_out(b):
            pltpu.make_async_copy(o_v.at[b],
                                  out_hbm.at[:, :, pl.ds(0, _Z)],
                                  so[b]).wait()

        def compute(b):
            a_vb = a_v.at[b]
            w_vb = w_v.at[b]
            o_vb = o_v.at[b]

            @plsc.parallel_loop(0, _Z, step=L)
            def pos_body(p):
                a_regs = [a_vb.at[i][pl.ds(p, L)] for i in range(_NIR)]
                for u in range(_MULT):
                    w_regs = [w_vb.at[u * _NWC + kk][pl.ds(p, L)]
                              for kk in range(_NWC)]
                    for i in range(_NIR):
                        o_vb.at[i].at[u][pl.ds(p, L)] = (
                            a_regs[i] * w_regs[_WIDX[i]])

        ncols = hi - lo
        start_in(lo, 0)

        def body(it, carry):
            for b in range(2):
                ci = it * 2 + b
                col = lo + ci

                @pl.when(ci < ncols)
                def _():
                    wait_in(b)

                    @pl.when(ci + 1 < ncols)
                    def _():
                        start_in(col + 1, 1 - b)

                    @pl.when(ci >= 2)
                    def _():
                        wait_out(b)

                    compute(b)
                    start_out(col, b)
            return carry

        lax.fori_loop(0, (ncols + 1) // 2, body, 0, unroll=False)

        even = (ncols % 2) == 0

        @pl.when(even)
        def _():
            wait_out(0)
            wait_out(1)

        @pl.when(jnp.logical_not(even))
        def _():
            wait_out(1)
            wait_out(0)

    return k


def kernel(edge_attr, weights):
    E = edge_attr.shape[0]
    a_t = edge_attr.T
    w_t = weights.T
    out_t = _build(E)(a_t, w_t)
    return out_t.transpose(2, 1, 0)
